# baseline (device time: 26855 ns/iter reference)
import jax
import jax.numpy as jnp
from jax import lax
from jax.experimental import pallas as pl
from jax.experimental.pallas import tpu as pltpu

N_DEV = 4
N_LAYERS = 3
N_PEERS = N_DEV - 1


def kernel(x, Win0, Wout0, Win1, Wout1, Win2, Wout2):
    b, d = x.shape
    hdim = Win0.shape[1]

    def body(x_ref, win0_ref, wout0_ref, win1_ref, wout1_ref, win2_ref,
             wout2_ref, out_ref, win_scr, wout_scr, send_buf, recv_buf,
             final_buf, w_sems, send_sems, recv_sems):
        my = lax.axis_index("i")
        peers = tuple(my ^ k for k in range(1, N_DEV))

        win_hbm = (win0_ref, win1_ref, win2_ref)
        wout_hbm = (wout0_ref, wout1_ref, wout2_ref)
        w_copies = []
        for l in range(N_LAYERS):
            cin = pltpu.make_async_copy(win_hbm[l], win_scr.at[l],
                                        w_sems.at[2 * l])
            cout = pltpu.make_async_copy(wout_hbm[l], wout_scr.at[l],
                                         w_sems.at[2 * l + 1])
            cin.start()
            cout.start()
            w_copies.append((cin, cout))

        barrier_sem = pltpu.get_barrier_semaphore()
        for nbr in peers:
            pl.semaphore_signal(
                barrier_sem, inc=1,
                device_id=(nbr,), device_id_type=pl.DeviceIdType.MESH,
            )
        pl.semaphore_wait(barrier_sem, N_PEERS)

        x_cur = x_ref[...].astype(jnp.bfloat16)
        acc = None
        for l in range(N_LAYERS):
            w_copies[l][0].wait()
            h = jnp.dot(x_cur, win_scr[l].astype(jnp.bfloat16),
                        preferred_element_type=jnp.float32)
            h = jnp.maximum(h, 0.0).astype(jnp.bfloat16)
            w_copies[l][1].wait()
            acc = jnp.dot(h, wout_scr[l].astype(jnp.bfloat16),
                          preferred_element_type=jnp.float32)
            send_buf[l] = acc.astype(jnp.bfloat16)
            rdmas = []
            for j in range(N_PEERS):
                slot = N_PEERS * l + j
                rdma = pltpu.make_async_remote_copy(
                    src_ref=send_buf.at[l],
                    dst_ref=recv_buf.at[slot],
                    send_sem=send_sems.at[slot],
                    recv_sem=recv_sems.at[slot],
                    device_id=(peers[j],),
                    device_id_type=pl.DeviceIdType.MESH,
                )
                rdma.start()
                rdmas.append(rdma)
            for rdma in rdmas:
                rdma.wait_recv()
            for j in range(N_PEERS):
                acc = acc + recv_buf[N_PEERS * l + j].astype(jnp.float32)
            for rdma in rdmas:
                rdma.wait_send()
            x_cur = acc.astype(jnp.bfloat16)

        rows = b // N_DEV
        final_buf[...] = acc
        out_ref[...] = final_buf[pl.ds(my * rows, rows), :]

    return pl.pallas_call(
        body,
        out_shape=jax.ShapeDtypeStruct((b // N_DEV, d), jnp.float32),
        in_specs=[pl.BlockSpec(memory_space=pltpu.VMEM)]
        + [pl.BlockSpec(memory_space=pl.ANY)] * 6,
        out_specs=pl.BlockSpec(memory_space=pltpu.VMEM),
        scratch_shapes=[
            pltpu.VMEM((N_LAYERS, d, hdim), jnp.float32),
            pltpu.VMEM((N_LAYERS, hdim, d), jnp.float32),
            pltpu.VMEM((N_LAYERS, b, d), jnp.bfloat16),
            pltpu.VMEM((N_LAYERS * N_PEERS, b, d), jnp.bfloat16),
            pltpu.VMEM((b, d), jnp.float32),
            pltpu.SemaphoreType.DMA((2 * N_LAYERS,)),
            pltpu.SemaphoreType.DMA((N_LAYERS * N_PEERS,)),
            pltpu.SemaphoreType.DMA((N_LAYERS * N_PEERS,)),
        ],
        compiler_params=pltpu.CompilerParams(collective_id=0),
    )(x, Win0, Wout0, Win1, Wout1, Win2, Wout2)


# device time: 21694 ns/iter; 1.2379x vs baseline; 1.2379x over previous
import jax
import jax.numpy as jnp
from jax import lax
from jax.experimental import pallas as pl
from jax.experimental.pallas import tpu as pltpu

N_DEV = 4
N_LAYERS = 3
N_PEERS = N_DEV - 1


def kernel(x, Win0, Wout0, Win1, Wout1, Win2, Wout2):
    b, d = x.shape

    rows = b // N_DEV

    def body(x_ref, win_ref, wout_ref, out_ref, send_buf, recv_buf,
             rs_recv_buf, send_sems, recv_sems):
        my = lax.axis_index("i")

        barrier_sem = pltpu.get_barrier_semaphore()
        for k in range(1, N_DEV):
            pl.semaphore_signal(
                barrier_sem, inc=1,
                device_id=(my ^ k,), device_id_type=pl.DeviceIdType.MESH,
            )

        issue_order = (2, 1, 3)
        arrive_order = (1, 3, 2)

        x_cur = x_ref[...]
        acc = None
        for l in range(N_LAYERS):
            h = jnp.dot(x_cur, win_ref[l],
                        preferred_element_type=jnp.float32)
            h = jnp.maximum(h, 0.0).astype(jnp.bfloat16)
            acc = jnp.dot(h, wout_ref[l],
                          preferred_element_type=jnp.float32)
            send_buf[l] = acc.astype(jnp.bfloat16)
            if l == 0:
                pl.semaphore_wait(barrier_sem, N_PEERS)
            last = l == N_LAYERS - 1
            rdmas = {}
            for k in issue_order:
                slot = N_PEERS * l + (k - 1)
                if last:
                    rdma = pltpu.make_async_remote_copy(
                        src_ref=send_buf.at[l, pl.ds((my ^ k) * rows, rows)],
                        dst_ref=rs_recv_buf.at[k - 1],
                        send_sem=send_sems.at[slot],
                        recv_sem=recv_sems.at[slot],
                        device_id=(my ^ k,),
                        device_id_type=pl.DeviceIdType.MESH,
                    )
                else:
                    rdma = pltpu.make_async_remote_copy(
                        src_ref=send_buf.at[l],
                        dst_ref=recv_buf.at[slot],
                        send_sem=send_sems.at[slot],
                        recv_sem=recv_sems.at[slot],
                        device_id=(my ^ k,),
                        device_id_type=pl.DeviceIdType.MESH,
                    )
                rdma.start()
                rdmas[k] = rdma
            if last:
                out = send_buf[l, pl.ds(my * rows, rows), :].astype(jnp.float32)
                for k in arrive_order:
                    rdmas[k].wait_recv()
                    out = out + rs_recv_buf[k - 1].astype(jnp.float32)
                out_ref[...] = out
            else:
                for k in arrive_order:
                    rdmas[k].wait_recv()
                    acc = acc + recv_buf[N_PEERS * l + (k - 1)].astype(
                        jnp.float32)
                x_cur = acc.astype(jnp.bfloat16)
            for k in issue_order:
                rdmas[k].wait_send()

    bf = jnp.bfloat16
    return pl.pallas_call(
        body,
        out_shape=jax.ShapeDtypeStruct((b // N_DEV, d), jnp.float32),
        in_specs=[pl.BlockSpec(memory_space=pltpu.VMEM)] * 3,
        out_specs=pl.BlockSpec(memory_space=pltpu.VMEM),
        scratch_shapes=[
            pltpu.VMEM((N_LAYERS, b, d), jnp.bfloat16),
            pltpu.VMEM((N_LAYERS * N_PEERS, b, d), jnp.bfloat16),
            pltpu.VMEM((N_PEERS, b // N_DEV, d), jnp.bfloat16),
            pltpu.SemaphoreType.DMA((N_LAYERS * N_PEERS,)),
            pltpu.SemaphoreType.DMA((N_LAYERS * N_PEERS,)),
        ],
        compiler_params=pltpu.CompilerParams(collective_id=0),
    )(x.astype(bf),
      jnp.stack([Win0, Win1, Win2]).astype(bf),
      jnp.stack([Wout0, Wout1, Wout2]).astype(bf))
